# Initial kernel scaffold; baseline (speedup 1.0000x reference)
import jax
import jax.numpy as jnp
from jax.experimental import pallas as pl


def kernel(x, edge_index, edge_attr, batch, We1, be1, W1a, b1a, W1b, b1b, We2, be2, W2a, b2a, W2b, b2b, Wl1, bl1, Wl2, bl2):
    raise NotImplementedError("write your pallas kernel here")


# speedup vs baseline: 1.8756x; 1.8756x over previous
import functools

import jax
import jax.numpy as jnp
from jax import lax
from jax.experimental import pallas as pl
from jax.experimental.pallas import tpu as pltpu
from jax.experimental.pallas import tpu_sc as plsc

N = 10000
NP = 10240
E = 320000
EPAD = 327680
F = 128
FH = 64
FE = 16
G = 64
NS = 16
NC = 2
ES = EPAD // NS
C = 128
CP = C // 2
NCH = ES // C
EPP = EPAD // 2
NROW = NP // NS
RB = 128
DUMP = N
BEP = 2048
BN = 1024


def _edge_lin_body(ea_ref, w1_ref, b1_ref, w2_ref, b2_ref, e1_ref, e2_ref):
    ea = ea_ref[...]
    e1_ref[...] = (jnp.dot(ea, w1_ref[0], preferred_element_type=jnp.float32)
                   + b1_ref[0])[None]
    e2_ref[...] = (jnp.dot(ea, w2_ref[0], preferred_element_type=jnp.float32)
                   + b2_ref[0])[None]


def _pack_w(We, be):
    z = jnp.zeros((FE, FH), jnp.float32)
    ws, bs = [], []
    for c in range(NC):
        wh = We[:, c * FH:(c + 1) * FH]
        bh = be[c * FH:(c + 1) * FH]
        top = jnp.concatenate([wh, z], axis=1)
        bot = jnp.concatenate([z, wh], axis=1)
        ws.append(jnp.concatenate([top, bot], axis=0))
        bs.append(jnp.concatenate([bh, bh]).reshape(1, F))
    return jnp.stack(ws), jnp.stack(bs)


def _edge_lin(edge_attr, We1, be1, We2, be2):
    w1p, b1p = _pack_w(We1, be1)
    w2p, b2p = _pack_w(We2, be2)
    ea2 = jnp.concatenate(
        [edge_attr.reshape(E // 2, 2 * FE),
         jnp.zeros((EPP - E // 2, 2 * FE), jnp.float32)])
    return pl.pallas_call(
        _edge_lin_body,
        grid=(EPP // BEP, NC),
        in_specs=[
            pl.BlockSpec((BEP, 2 * FE), lambda i, c: (i, 0)),
            pl.BlockSpec((1, 2 * FE, F), lambda i, c: (c, 0, 0)),
            pl.BlockSpec((1, 1, F), lambda i, c: (c, 0, 0)),
            pl.BlockSpec((1, 2 * FE, F), lambda i, c: (c, 0, 0)),
            pl.BlockSpec((1, 1, F), lambda i, c: (c, 0, 0)),
        ],
        out_specs=[
            pl.BlockSpec((1, BEP, F), lambda i, c: (c, i, 0)),
            pl.BlockSpec((1, BEP, F), lambda i, c: (c, i, 0)),
        ],
        out_shape=[
            jax.ShapeDtypeStruct((NC, EPP, F), jnp.float32),
            jax.ShapeDtypeStruct((NC, EPP, F), jnp.float32),
        ],
    )(ea2, w1p, b1p, w2p, b2p)


def _gine_agg_body(xs_hbm, e_hbm, src_hbm, dst_hbm, zer_hbm, out_hbm,
                   x_sh, agg_sh, src_v, dst_v, rows_v, e_v, bounce_v,
                   sem1, sem2, sem3):
    c = lax.axis_index("c")
    s = lax.axis_index("s")

    pltpu.sync_copy(zer_hbm, bounce_v)
    for k in range(NROW // RB):
        r0 = s * NROW + k * RB
        pltpu.sync_copy(bounce_v, agg_sh.at[pl.ds(r0, RB)])
    for k in range(NROW // RB):
        r0 = s * NROW + k * RB
        pltpu.sync_copy(xs_hbm.at[c, pl.ds(r0, RB)], bounce_v)
        pltpu.sync_copy(bounce_v, x_sh.at[pl.ds(r0, RB)])
    plsc.subcore_barrier()

    def chunk_body(j, carry):
        basep = s * (ES // 2) + j * CP
        cp_e = pltpu.async_copy(e_hbm.at[c, pl.ds(basep, CP)], e_v, sem1)
        cp_s = pltpu.async_copy(src_hbm.at[s, j], src_v, sem3)
        pltpu.sync_copy(dst_hbm.at[s, j], dst_v)
        cp_s.wait()
        cp_g = pltpu.async_copy(x_sh.at[src_v], rows_v, sem2)
        cp_e.wait()
        cp_g.wait()

        def row_body(r2, rcarry):
            for k2 in range(F // 16):
                r = 2 * r2 + (k2 // 4)
                sl = pl.ds((k2 % 4) * 16, 16)
                rows_v[r, sl] = jnp.maximum(
                    rows_v[r, sl] + e_v[r2, pl.ds(k2 * 16, 16)], 0.0)
            return rcarry

        lax.fori_loop(0, CP, row_body, 0, unroll=2)
        pltpu.sync_copy(rows_v, agg_sh.at[dst_v], add=True)
        return carry

    lax.fori_loop(0, NCH, chunk_body, 0)
    plsc.subcore_barrier()

    for k in range(NROW // RB):
        r0 = s * NROW + k * RB
        pltpu.sync_copy(agg_sh.at[pl.ds(r0, RB)], bounce_v)
        pltpu.sync_copy(bounce_v, out_hbm.at[c, pl.ds(r0, RB)])


@functools.lru_cache(maxsize=None)
def _gine_agg_kernel():
    return pl.kernel(
        _gine_agg_body,
        out_type=jax.ShapeDtypeStruct((NC, NP, FH), jnp.float32),
        mesh=plsc.VectorSubcoreMesh(core_axis_name="c", subcore_axis_name="s",
                                    num_cores=NC, num_subcores=NS),
        scratch_types=[
            pltpu.VMEM_SHARED((NP, FH), jnp.float32),
            pltpu.VMEM_SHARED((NP, FH), jnp.float32),
            pltpu.VMEM((C,), jnp.int32),
            pltpu.VMEM((C,), jnp.int32),
            pltpu.VMEM((C, FH), jnp.float32),
            pltpu.VMEM((CP, F), jnp.float32),
            pltpu.VMEM((RB, FH), jnp.float32),
            pltpu.SemaphoreType.DMA,
            pltpu.SemaphoreType.DMA,
            pltpu.SemaphoreType.DMA,
        ],
    )


def _gine_agg(xs, e, src3, dst3, zer):
    return _gine_agg_kernel()(xs, e, src3, dst3, zer)


def _elu(v):
    return jnp.where(v > 0, v, jnp.exp(v) - 1.0)


def _node_mlp_body(x_ref, agg_ref, wa_ref, ba_ref, wb_ref, bb_ref, out_ref):
    t = jnp.concatenate([x_ref[0] + agg_ref[0], x_ref[1] + agg_ref[1]],
                        axis=-1)
    u = _elu(jnp.dot(t, wa_ref[...], preferred_element_type=jnp.float32)
             + ba_ref[...])
    v = jnp.dot(u, wb_ref[...], preferred_element_type=jnp.float32) + bb_ref[...]
    w = _elu(v)
    out_ref[0] = w[:, :FH]
    out_ref[1] = w[:, FH:]


def _node_mlp(xs, agg, Wa, ba, Wb, bb):
    return pl.pallas_call(
        _node_mlp_body,
        grid=(NP // BN,),
        in_specs=[
            pl.BlockSpec((NC, BN, FH), lambda i: (0, i, 0)),
            pl.BlockSpec((NC, BN, FH), lambda i: (0, i, 0)),
            pl.BlockSpec((F, F), lambda i: (0, 0)),
            pl.BlockSpec((1, F), lambda i: (0, 0)),
            pl.BlockSpec((F, F), lambda i: (0, 0)),
            pl.BlockSpec((1, F), lambda i: (0, 0)),
        ],
        out_specs=pl.BlockSpec((NC, BN, FH), lambda i: (0, i, 0)),
        out_shape=jax.ShapeDtypeStruct((NC, NP, FH), jnp.float32),
    )(xs, agg, Wa, ba.reshape(1, F), Wb, bb.reshape(1, F))


def _pool_head_body(h_ref, batch_ref, wl1_ref, bl1_ref, wl2_ref, bl2_ref,
                    out_ref, hg_ref):
    i = pl.program_id(0)

    @pl.when(i == 0)
    def _():
        hg_ref[...] = jnp.zeros((G, F), jnp.float32)

    h_blk = jnp.concatenate([h_ref[0], h_ref[1]], axis=-1)
    iota_g = lax.broadcasted_iota(jnp.int32, (G, BN), 0)
    onehot = (iota_g == batch_ref[0]).astype(jnp.float32)
    hg_ref[...] += jnp.dot(onehot, h_blk, preferred_element_type=jnp.float32)

    @pl.when(i == NP // BN - 1)
    def _():
        hg2 = jnp.maximum(
            jnp.dot(hg_ref[...], wl1_ref[...],
                    preferred_element_type=jnp.float32) + bl1_ref[...], 0.0)
        out_ref[...] = (jnp.dot(hg2, wl2_ref[...],
                                preferred_element_type=jnp.float32)
                        + bl2_ref[...])


def _pool_head(hs, batch3, Wl1, bl1, Wl2, bl2):
    return pl.pallas_call(
        _pool_head_body,
        grid=(NP // BN,),
        in_specs=[
            pl.BlockSpec((NC, BN, FH), lambda i: (0, i, 0)),
            pl.BlockSpec((1, 1, BN), lambda i: (i, 0, 0)),
            pl.BlockSpec((F, F), lambda i: (0, 0)),
            pl.BlockSpec((1, F), lambda i: (0, 0)),
            pl.BlockSpec((F, 1), lambda i: (0, 0)),
            pl.BlockSpec((1, 1), lambda i: (0, 0)),
        ],
        out_specs=pl.BlockSpec((G, 1), lambda i: (0, 0)),
        out_shape=jax.ShapeDtypeStruct((G, 1), jnp.float32),
        scratch_shapes=[pltpu.VMEM((G, F), jnp.float32)],
    )(hs, batch3, Wl1, bl1.reshape(1, F), Wl2, bl2.reshape(1, 1))


def kernel(x, edge_index, edge_attr, batch,
           We1, be1, W1a, b1a, W1b, b1b,
           We2, be2, W2a, b2a, W2b, b2b,
           Wl1, bl1, Wl2, bl2):
    src3 = jnp.concatenate(
        [edge_index[0].astype(jnp.int32),
         jnp.zeros((EPAD - E,), jnp.int32)]).reshape(NS, NCH, C)
    dst3 = jnp.concatenate(
        [edge_index[1].astype(jnp.int32),
         jnp.full((EPAD - E,), DUMP, jnp.int32)]).reshape(NS, NCH, C)
    xs = jnp.stack([x[:, :FH], x[:, FH:]])
    xs = jnp.concatenate(
        [xs, jnp.zeros((NC, NP - N, FH), jnp.float32)], axis=1)
    zer = jnp.zeros((RB, FH), jnp.float32)
    batch3 = jnp.concatenate(
        [batch.astype(jnp.int32), jnp.full((NP - N,), G, jnp.int32)]
    ).reshape(NP // BN, 1, BN)

    e1, e2 = _edge_lin(edge_attr, We1, be1, We2, be2)

    agg1 = _gine_agg(xs, e1, src3, dst3, zer)
    hs = _node_mlp(xs, agg1, W1a, b1a, W1b, b1b)

    agg2 = _gine_agg(hs, e2, src3, dst3, zer)
    hs = _node_mlp(hs, agg2, W2a, b2a, W2b, b2b)

    return _pool_head(hs, batch3, Wl1, bl1, Wl2, bl2)
